# Initial kernel scaffold; baseline (speedup 1.0000x reference)
import jax
import jax.numpy as jnp
from jax.experimental import pallas as pl


def kernel(in_feat, edge_index1, etype1, edge_index2, etype2, edge_index3, etype3, edge_index4, etype4, edge_index5, etype5, params):
    raise NotImplementedError("write your pallas kernel here")


# speedup vs baseline: 2.3120x; 2.3120x over previous
import functools

import jax
import jax.numpy as jnp
from jax import lax
from jax.experimental import pallas as pl
from jax.experimental.pallas import tpu as pltpu
from jax.experimental.pallas import tpu_sc as plsc

_NSC = 2
_NTILE = 16


def _block_rows(n, cap):
    b = min(n, cap)
    while n % b:
        b //= 2
    return b


def _mm_body(coeff_ref, b0_ref, b1_ref, ws_ref, h_ref, z_ref, s_ref):
    h = h_ref[...]
    hb0 = jnp.dot(h, b0_ref[...], preferred_element_type=jnp.float32)
    hb1 = jnp.dot(h, b1_ref[...], preferred_element_type=jnp.float32)
    s_ref[...] = jnp.dot(h, ws_ref[...], preferred_element_type=jnp.float32)
    c = coeff_ref[...]
    parts = [c[r, 0] * hb0 + c[r, 1] * hb1 for r in range(9)]
    z_ref[...] = jnp.concatenate(parts, axis=1)


def _rel_matmul(h, coeff, bases, wself):
    n, cin = h.shape
    cout = wself.shape[1]
    bn = _block_rows(n, 2048)
    grid = (n // bn,)
    z, s = pl.pallas_call(
        _mm_body,
        grid=grid,
        in_specs=[
            pl.BlockSpec((9, 2), lambda i: (0, 0)),
            pl.BlockSpec((cin, cout), lambda i: (0, 0)),
            pl.BlockSpec((cin, cout), lambda i: (0, 0)),
            pl.BlockSpec((cin, cout), lambda i: (0, 0)),
            pl.BlockSpec((bn, cin), lambda i: (i, 0)),
        ],
        out_specs=[
            pl.BlockSpec((bn, 9 * cout), lambda i: (i, 0)),
            pl.BlockSpec((bn, cout), lambda i: (i, 0)),
        ],
        out_shape=[
            jax.ShapeDtypeStruct((n, 9 * cout), jnp.float32),
            jax.ShapeDtypeStruct((n, cout), jnp.float32),
        ],
    )(coeff, bases[0], bases[1], wself, h)
    return z, s


def _combine_body(p0_ref, p1_ref, s_ref, b_ref, o_ref, *, do_relu):
    x = p0_ref[...] + p1_ref[...] + s_ref[...] + b_ref[...]
    o_ref[...] = jnp.maximum(x, 0.0) if do_relu else x


def _combine(p0, p1, s, bias, do_relu):
    n, cout = s.shape
    bn = _block_rows(n, 4096)
    grid = (n // bn,)
    return pl.pallas_call(
        functools.partial(_combine_body, do_relu=do_relu),
        grid=grid,
        in_specs=[
            pl.BlockSpec((bn, cout), lambda i: (i, 0)),
            pl.BlockSpec((bn, cout), lambda i: (i, 0)),
            pl.BlockSpec((bn, cout), lambda i: (i, 0)),
            pl.BlockSpec((1, cout), lambda i: (0, 0)),
        ],
        out_specs=pl.BlockSpec((bn, cout), lambda i: (i, 0)),
        out_shape=jax.ShapeDtypeStruct((n, cout), jnp.float32),
    )(p0, p1, s, bias.reshape(1, cout))


def _pool_body(x_ref, o_ref, *, res):
    c = x_ref.shape[-1]
    x = x_ref[...].reshape(res // 2, 2, res // 2, 2, c)
    o_ref[...] = x.mean(axis=(1, 3)).reshape((res // 2) * (res // 2), c)


def _meanpool(h, res):
    n, c = h.shape
    face = res * res
    return pl.pallas_call(
        functools.partial(_pool_body, res=res),
        grid=(6,),
        in_specs=[pl.BlockSpec((face, c), lambda t: (t, 0))],
        out_specs=pl.BlockSpec((face // 4, c), lambda t: (t, 0)),
        out_shape=jax.ShapeDtypeStruct((n // 4, c), jnp.float32),
    )(h)


def _up_body(w_ref, b_ref, x_ref, o_ref):
    o_ref[...] = (
        jnp.dot(x_ref[...], w_ref[...], preferred_element_type=jnp.float32)
        + b_ref[...]
    )


def _upsample(h, res, p):
    n, c = h.shape
    d = p['w'].shape[1]
    wcat = jnp.transpose(p['w'], (0, 2, 3, 1)).reshape(c, 4 * d)
    btile = jnp.tile(p['b'], 4).reshape(1, 4 * d)
    bn = _block_rows(n, 2048)
    y = pl.pallas_call(
        _up_body,
        grid=(n // bn,),
        in_specs=[
            pl.BlockSpec((c, 4 * d), lambda i: (0, 0)),
            pl.BlockSpec((1, 4 * d), lambda i: (0, 0)),
            pl.BlockSpec((bn, c), lambda i: (i, 0)),
        ],
        out_specs=pl.BlockSpec((bn, 4 * d), lambda i: (i, 0)),
        out_shape=jax.ShapeDtypeStruct((n, 4 * d), jnp.float32),
    )(wcat, btile, h)
    y = y.reshape(6, res, res, 2, 2, d)
    y = y.transpose(0, 1, 3, 2, 4, 5).reshape(6 * 4 * res * res, d)
    return y


@functools.lru_cache(maxsize=None)
def _make_seg_kernel(n, e, cout):
    n_tiles = _NSC * _NTILE
    per_tile = e // n_tiles
    k = 128 if per_tile % 128 == 0 else per_tile
    n_chunks = per_tile // k
    rows_pt = n // _NTILE
    mesh = plsc.VectorSubcoreMesh(core_axis_name="c", subcore_axis_name="s")

    @functools.partial(
        pl.kernel,
        out_type=jax.ShapeDtypeStruct((_NSC, n, cout), jnp.float32),
        mesh=mesh,
        compiler_params=pltpu.CompilerParams(use_tc_tiling_on_sc=False),
        scratch_types=[
            pltpu.VMEM((1, k), jnp.int32),
            pltpu.VMEM((1, k), jnp.int32),
            pltpu.VMEM((1, k), jnp.int32),
            pltpu.VMEM((1, k), jnp.int32),
            pltpu.VMEM((k, cout), jnp.float32),
            pltpu.VMEM_SHARED((n, cout), jnp.float32),
            pltpu.SemaphoreType.DMA,
        ],
    )
    def seg(zt_hbm, src_hbm, et_hbm, dst_hbm, zeros_hbm, out_hbm,
            src_v, et_v, dst_v, flat_v, rows_v, acc, sem):
        cid = lax.axis_index("c")
        sid = lax.axis_index("s")
        r0 = sid * rows_pt
        pltpu.sync_copy(zeros_hbm.at[pl.ds(r0, rows_pt)],
                        acc.at[pl.ds(r0, rows_pt)])
        plsc.subcore_barrier()

        eb = (cid * _NTILE + sid) * per_tile

        def chunk(j, carry):
            off = eb + j * k
            pltpu.sync_copy(src_hbm.at[pl.ds(off, k)], src_v.at[0])
            pltpu.sync_copy(et_hbm.at[pl.ds(off, k)], et_v.at[0])
            pltpu.sync_copy(dst_hbm.at[pl.ds(off, k)], dst_v.at[0])
            for t in range(k // 16):
                sl = pl.ds(t * 16, 16)
                flat_v[0, sl] = src_v[0, sl] * 9 + et_v[0, sl]
            pltpu.async_copy(zt_hbm.at[flat_v.at[0]], rows_v, sem).wait()
            pltpu.sync_copy(rows_v, acc.at[dst_v.at[0]], add=True)
            return carry

        lax.fori_loop(0, n_chunks, chunk, 0)
        plsc.subcore_barrier()
        pltpu.sync_copy(acc.at[pl.ds(r0, rows_pt)],
                        out_hbm.at[cid, pl.ds(r0, rows_pt)])

    return seg


def _rel_conv(h, src, dst, et, zeros, p, n, do_relu):
    cout = p['wself'].shape[1]
    z, s = _rel_matmul(h, p['coeff'], p['bases'], p['wself'])
    zt = z.reshape(n * 9, cout)
    part = _make_seg_kernel(n, src.shape[0], cout)(zt, src, et, dst, zeros)
    return _combine(part[0], part[1], s, p['bias'], do_relu)


def kernel(in_feat, edge_index1, etype1, edge_index2, etype2, edge_index3,
           etype3, edge_index4, etype4, edge_index5, etype5, params):
    res = 128
    ns = [6 * (res // 2 ** l) ** 2 for l in range(5)]
    edges = []
    for ei, et in ((edge_index1, etype1), (edge_index2, etype2),
                   (edge_index3, etype3), (edge_index4, etype4),
                   (edge_index5, etype5)):
        edges.append((ei[0], ei[1], et))

    zero_cache = {}

    def rc(h, level, name, do_relu=True):
        s, d, t = edges[level]
        n = ns[level]
        cout = params[name]['wself'].shape[1]
        zk = (n, cout)
        if zk not in zero_cache:
            zero_cache[zk] = jnp.zeros((n, cout), jnp.float32)
        return _rel_conv(h, s, d, t, zero_cache[zk], params[name], n, do_relu)

    h1 = rc(in_feat, 0, 'conv1')
    h22 = rc(h1, 0, 'conv2')
    h2 = _meanpool(h22, res)
    h3 = rc(h2, 1, 'conv3')
    h33 = rc(h3, 1, 'conv33')
    h4i = _meanpool(h33, res // 2)
    h4 = rc(h4i, 2, 'conv4')
    h44 = rc(h4, 2, 'conv44')
    h5i = _meanpool(h44, res // 4)
    h5 = rc(h5i, 3, 'conv5')
    h55 = rc(h5, 3, 'conv55')
    h6i = _meanpool(h55, res // 8)
    h6 = rc(h6i, 4, 'conv6')
    h6 = rc(h6, 4, 'conv66')
    h6 = rc(h6, 4, 'conv666')
    h6 = _upsample(h6, res // 16, params['up1'])
    h6 = jnp.concatenate([h6, h55], axis=1)
    h6 = rc(h6, 3, 'conv7')
    h6 = rc(h6, 3, 'conv77')
    h6 = rc(h6, 3, 'conv777')
    h6 = _upsample(h6, res // 8, params['up2'])
    h6 = jnp.concatenate([h6, h44], axis=1)
    h6 = rc(h6, 2, 'conv8')
    h6 = rc(h6, 2, 'conv88')
    h6 = rc(h6, 2, 'conv888')
    h6 = _upsample(h6, res // 4, params['up3'])
    h6 = jnp.concatenate([h6, h33], axis=1)
    h6 = rc(h6, 1, 'conv9')
    h6 = rc(h6, 1, 'conv99')
    h6 = rc(h6, 1, 'conv999')
    h6 = _upsample(h6, res // 2, params['up4'])
    h6 = jnp.concatenate([h6, h22], axis=1)
    h6 = rc(h6, 0, 'conv10')
    h6 = rc(h6, 0, 'conv101')
    out = rc(h6, 0, 'conv11', do_relu=False)
    return out


# speedup vs baseline: 2.8127x; 1.2166x over previous
import functools

import jax
import jax.numpy as jnp
from jax import lax
from jax.experimental import pallas as pl
from jax.experimental.pallas import tpu as pltpu
from jax.experimental.pallas import tpu_sc as plsc

_NSC = 2
_NTILE = 16


def _block_rows(n, cap):
    b = min(n, cap)
    while n % b:
        b //= 2
    return b


def _mm_body(coeff_ref, b0_ref, b1_ref, ws_ref, h_ref, z_ref, s_ref):
    h = h_ref[...]
    hb0 = jnp.dot(h, b0_ref[...], preferred_element_type=jnp.float32)
    hb1 = jnp.dot(h, b1_ref[...], preferred_element_type=jnp.float32)
    s_ref[...] = jnp.dot(h, ws_ref[...], preferred_element_type=jnp.float32)
    c = coeff_ref[...]
    parts = [c[r, 0] * hb0 + c[r, 1] * hb1 for r in range(9)]
    z_ref[...] = jnp.concatenate(parts, axis=1)


def _rel_matmul(h, coeff, bases, wself):
    n, cin = h.shape
    cout = wself.shape[1]
    bn = _block_rows(n, 2048)
    grid = (n // bn,)
    z, s = pl.pallas_call(
        _mm_body,
        grid=grid,
        in_specs=[
            pl.BlockSpec((9, 2), lambda i: (0, 0)),
            pl.BlockSpec((cin, cout), lambda i: (0, 0)),
            pl.BlockSpec((cin, cout), lambda i: (0, 0)),
            pl.BlockSpec((cin, cout), lambda i: (0, 0)),
            pl.BlockSpec((bn, cin), lambda i: (i, 0)),
        ],
        out_specs=[
            pl.BlockSpec((bn, 9 * cout), lambda i: (i, 0)),
            pl.BlockSpec((bn, cout), lambda i: (i, 0)),
        ],
        out_shape=[
            jax.ShapeDtypeStruct((n, 9 * cout), jnp.float32),
            jax.ShapeDtypeStruct((n, cout), jnp.float32),
        ],
    )(coeff, bases[0], bases[1], wself, h)
    return z, s


def _combine_body(p0_ref, p1_ref, s_ref, b_ref, o_ref, *, do_relu):
    x = p0_ref[...] + p1_ref[...] + s_ref[...] + b_ref[...]
    o_ref[...] = jnp.maximum(x, 0.0) if do_relu else x


def _combine(p0, p1, s, bias, do_relu):
    n, cout = s.shape
    bn = _block_rows(n, 4096)
    grid = (n // bn,)
    return pl.pallas_call(
        functools.partial(_combine_body, do_relu=do_relu),
        grid=grid,
        in_specs=[
            pl.BlockSpec((bn, cout), lambda i: (i, 0)),
            pl.BlockSpec((bn, cout), lambda i: (i, 0)),
            pl.BlockSpec((bn, cout), lambda i: (i, 0)),
            pl.BlockSpec((1, cout), lambda i: (0, 0)),
        ],
        out_specs=pl.BlockSpec((bn, cout), lambda i: (i, 0)),
        out_shape=jax.ShapeDtypeStruct((n, cout), jnp.float32),
    )(p0, p1, s, bias.reshape(1, cout))


def _pool_body(x_ref, o_ref, *, res):
    c = x_ref.shape[-1]
    x = x_ref[...].reshape(res // 2, 2, res // 2, 2, c)
    o_ref[...] = x.mean(axis=(1, 3)).reshape((res // 2) * (res // 2), c)


def _meanpool(h, res):
    n, c = h.shape
    face = res * res
    return pl.pallas_call(
        functools.partial(_pool_body, res=res),
        grid=(6,),
        in_specs=[pl.BlockSpec((face, c), lambda t: (t, 0))],
        out_specs=pl.BlockSpec((face // 4, c), lambda t: (t, 0)),
        out_shape=jax.ShapeDtypeStruct((n // 4, c), jnp.float32),
    )(h)


def _up_body(w_ref, b_ref, x_ref, o_ref):
    o_ref[...] = (
        jnp.dot(x_ref[...], w_ref[...], preferred_element_type=jnp.float32)
        + b_ref[...]
    )


def _upsample(h, res, p):
    n, c = h.shape
    d = p['w'].shape[1]
    wcat = jnp.transpose(p['w'], (0, 2, 3, 1)).reshape(c, 4 * d)
    btile = jnp.tile(p['b'], 4).reshape(1, 4 * d)
    bn = _block_rows(n, 2048)
    y = pl.pallas_call(
        _up_body,
        grid=(n // bn,),
        in_specs=[
            pl.BlockSpec((c, 4 * d), lambda i: (0, 0)),
            pl.BlockSpec((1, 4 * d), lambda i: (0, 0)),
            pl.BlockSpec((bn, c), lambda i: (i, 0)),
        ],
        out_specs=pl.BlockSpec((bn, 4 * d), lambda i: (i, 0)),
        out_shape=jax.ShapeDtypeStruct((n, 4 * d), jnp.float32),
    )(wcat, btile, h)
    y = y.reshape(6, res, res, 2, 2, d)
    y = y.transpose(0, 1, 3, 2, 4, 5).reshape(6 * 4 * res * res, d)
    return y


@functools.lru_cache(maxsize=None)
def _make_seg_kernel(n, e, cout):
    n_tiles = _NSC * _NTILE
    per_tile = e // n_tiles
    k = 128 if per_tile % 128 == 0 else per_tile
    sup = k
    for cand in (1024, 512, 384, 256, 128):
        if cand % k == 0 and per_tile % cand == 0:
            sup = cand
            break
    nb = sup // k
    n_chunks = per_tile // sup
    rows_pt = n // _NTILE
    mesh = plsc.VectorSubcoreMesh(core_axis_name="c", subcore_axis_name="s")

    @functools.partial(
        pl.kernel,
        out_type=jax.ShapeDtypeStruct((_NSC, n, cout), jnp.float32),
        mesh=mesh,
        compiler_params=pltpu.CompilerParams(use_tc_tiling_on_sc=False),
        scratch_types=[
            pltpu.VMEM((sup,), jnp.int32),
            pltpu.VMEM((sup,), jnp.int32),
            pltpu.VMEM((nb, k), jnp.int32),
            pltpu.VMEM((sup,), jnp.int32),
            pltpu.VMEM((2, k, cout), jnp.float32),
            pltpu.VMEM_SHARED((n, cout), jnp.float32),
            pltpu.SemaphoreType.DMA,
        ],
    )
    def seg(zt_hbm, src_hbm, et_hbm, dst_hbm, zeros_hbm, out_hbm,
            src_v, et_v, dst_v, flat_v, rows_v, acc, sem):
        cid = lax.axis_index("c")
        sid = lax.axis_index("s")
        r0 = sid * rows_pt
        pltpu.sync_copy(zeros_hbm.at[pl.ds(r0, rows_pt)],
                        acc.at[pl.ds(r0, rows_pt)])
        plsc.subcore_barrier()

        eb = (cid * _NTILE + sid) * per_tile

        def chunk(j, carry):
            off = eb + j * sup
            pltpu.sync_copy(src_hbm.at[pl.ds(off, sup)], src_v)
            pltpu.sync_copy(et_hbm.at[pl.ds(off, sup)], et_v)
            for b in range(nb):
                pltpu.sync_copy(dst_hbm.at[pl.ds(off + b * k, k)],
                                dst_v.at[b])
            for t in range(sup // 16):
                sl = pl.ds(t * 16, 16)
                flat_v[sl] = src_v[sl] * 9 + et_v[sl]
            hnd = pltpu.async_copy(zt_hbm.at[flat_v.at[pl.ds(0, k)]],
                                   rows_v.at[0], sem)
            for b in range(nb):
                hnd.wait()
                if b + 1 < nb:
                    hnd = pltpu.async_copy(
                        zt_hbm.at[flat_v.at[pl.ds((b + 1) * k, k)]],
                        rows_v.at[(b + 1) % 2], sem)
                pltpu.sync_copy(rows_v.at[b % 2], acc.at[dst_v.at[b]],
                                add=True)
            return carry

        lax.fori_loop(0, n_chunks, chunk, 0)
        plsc.subcore_barrier()
        pltpu.sync_copy(acc.at[pl.ds(r0, rows_pt)],
                        out_hbm.at[cid, pl.ds(r0, rows_pt)])

    return seg


def _rel_conv(h, src, dst, et, zeros, p, n, do_relu):
    cout = p['wself'].shape[1]
    z, s = _rel_matmul(h, p['coeff'], p['bases'], p['wself'])
    zt = z.reshape(n * 9, cout)
    part = _make_seg_kernel(n, src.shape[0], cout)(zt, src, et, dst, zeros)
    return _combine(part[0], part[1], s, p['bias'], do_relu)


def kernel(in_feat, edge_index1, etype1, edge_index2, etype2, edge_index3,
           etype3, edge_index4, etype4, edge_index5, etype5, params):
    res = 128
    ns = [6 * (res // 2 ** l) ** 2 for l in range(5)]
    edges = []
    for ei, et in ((edge_index1, etype1), (edge_index2, etype2),
                   (edge_index3, etype3), (edge_index4, etype4),
                   (edge_index5, etype5)):
        edges.append((ei[0], ei[1], et))

    zero_cache = {}

    def rc(h, level, name, do_relu=True):
        s, d, t = edges[level]
        n = ns[level]
        cout = params[name]['wself'].shape[1]
        zk = (n, cout)
        if zk not in zero_cache:
            zero_cache[zk] = jnp.zeros((n, cout), jnp.float32)
        return _rel_conv(h, s, d, t, zero_cache[zk], params[name], n, do_relu)

    h1 = rc(in_feat, 0, 'conv1')
    h22 = rc(h1, 0, 'conv2')
    h2 = _meanpool(h22, res)
    h3 = rc(h2, 1, 'conv3')
    h33 = rc(h3, 1, 'conv33')
    h4i = _meanpool(h33, res // 2)
    h4 = rc(h4i, 2, 'conv4')
    h44 = rc(h4, 2, 'conv44')
    h5i = _meanpool(h44, res // 4)
    h5 = rc(h5i, 3, 'conv5')
    h55 = rc(h5, 3, 'conv55')
    h6i = _meanpool(h55, res // 8)
    h6 = rc(h6i, 4, 'conv6')
    h6 = rc(h6, 4, 'conv66')
    h6 = rc(h6, 4, 'conv666')
    h6 = _upsample(h6, res // 16, params['up1'])
    h6 = jnp.concatenate([h6, h55], axis=1)
    h6 = rc(h6, 3, 'conv7')
    h6 = rc(h6, 3, 'conv77')
    h6 = rc(h6, 3, 'conv777')
    h6 = _upsample(h6, res // 8, params['up2'])
    h6 = jnp.concatenate([h6, h44], axis=1)
    h6 = rc(h6, 2, 'conv8')
    h6 = rc(h6, 2, 'conv88')
    h6 = rc(h6, 2, 'conv888')
    h6 = _upsample(h6, res // 4, params['up3'])
    h6 = jnp.concatenate([h6, h33], axis=1)
    h6 = rc(h6, 1, 'conv9')
    h6 = rc(h6, 1, 'conv99')
    h6 = rc(h6, 1, 'conv999')
    h6 = _upsample(h6, res // 2, params['up4'])
    h6 = jnp.concatenate([h6, h22], axis=1)
    h6 = rc(h6, 0, 'conv10')
    h6 = rc(h6, 0, 'conv101')
    out = rc(h6, 0, 'conv11', do_relu=False)
    return out
